# Initial kernel scaffold; baseline (speedup 1.0000x reference)
import jax
import jax.numpy as jnp
from jax.experimental import pallas as pl


def kernel(x, edge_index, lin_W, lin_b, conv1_W, conv1_b, bn1_gamma, bn1_beta, conv2_W, conv2_b):
    raise NotImplementedError("write your pallas kernel here")


# speedup vs baseline: 6.5403x; 6.5403x over previous
import functools

import jax
import jax.numpy as jnp
from jax import lax
from jax.experimental import pallas as pl
from jax.experimental.pallas import tpu as pltpu
from jax.experimental.pallas import tpu_sc as plsc

N = 10000
D = 128
E = 320000
NC = 2
NS = 16
NW = NC * NS
C = 128
CHUNKS = 79
EPW = C * CHUNKS
E_PAD = NW * EPW
NH = 5000
AR = 5008
RPA = 312
RPA_LAST_W = NH - (NS - 1) * RPA
RPA_LAST_I = AR - (NS - 1) * RPA
DEG_PAD = EPW
DPS = 632
BR = 1000
G = N // BR


def _sc_mesh():
    return plsc.VectorSubcoreMesh(core_axis_name="c", subcore_axis_name="s")


def _sc_deg(dst3, zeros1d, ones1d):
    @functools.partial(
        pl.kernel,
        mesh=_sc_mesh(),
        out_type=jax.ShapeDtypeStruct((NC, NS, DPS), jnp.float32),
        scratch_types=[
            pltpu.VMEM((CHUNKS, C), jnp.int32),
            pltpu.VMEM((C,), jnp.float32),
            pltpu.VMEM((DPS,), jnp.float32),
            pltpu.VMEM_SHARED((DEG_PAD,), jnp.float32),
        ],
    )
    def k(dst_hbm, z_hbm, ones_hbm, out_hbm, dst_v, ones_v, buf_v, deg_sh):
        c = lax.axis_index("c")
        s = lax.axis_index("s")
        wid = s * NC + c
        pltpu.sync_copy(dst_hbm.at[wid], dst_v)
        pltpu.sync_copy(ones_hbm, ones_v)
        pltpu.sync_copy(z_hbm, buf_v)
        pltpu.sync_copy(buf_v, deg_sh.at[pl.ds(s * DPS, DPS)])
        plsc.subcore_barrier()

        def body(j, carry):
            pltpu.sync_copy(ones_v, deg_sh.at[dst_v.at[j]], add=True)
            return carry

        lax.fori_loop(0, CHUNKS, body, 0)
        plsc.subcore_barrier()
        pltpu.sync_copy(deg_sh.at[pl.ds(s * DPS, DPS)], buf_v)
        pltpu.sync_copy(buf_v, out_hbm.at[c, s])

    return k(dst3, zeros1d, ones1d)


def _sc_agg(y, src3, dst3, zeros2d):
    @functools.partial(
        pl.kernel,
        mesh=_sc_mesh(),
        out_type=jax.ShapeDtypeStruct((2, NC, NH, D), jnp.float32),
        scratch_types=[
            pltpu.VMEM((CHUNKS, C), jnp.int32),
            pltpu.VMEM((CHUNKS, C), jnp.int32),
            pltpu.VMEM((C,), jnp.int32),
            pltpu.VMEM((C, D), jnp.float32),
            pltpu.VMEM((RPA_LAST_I, D), jnp.float32),
            pltpu.VMEM_SHARED((AR, D), jnp.float32),
            pltpu.SemaphoreType.DMA,
        ],
    )
    def k(y_hbm, src_hbm, dst_hbm, z_hbm, out_hbm, src_v, dst_v, midx_v,
          rows_v, stage_v, acc_sh, sem):
        c = lax.axis_index("c")
        s = lax.axis_index("s")
        pltpu.sync_copy(src_hbm.at[c * NS + s], src_v)
        pltpu.sync_copy(dst_hbm.at[c * NS + s], dst_v)

        for p in (0, 1):
            pltpu.sync_copy(z_hbm, stage_v)
            @pl.when(s < NS - 1)
            def _():
                pltpu.sync_copy(stage_v.at[pl.ds(0, RPA)],
                                acc_sh.at[pl.ds(s * RPA, RPA)])

            @pl.when(s == NS - 1)
            def _():
                pltpu.sync_copy(stage_v,
                                acc_sh.at[pl.ds((NS - 1) * RPA, RPA_LAST_I)])

            plsc.subcore_barrier()

            def body(j, carry):
                pltpu.async_copy(y_hbm.at[src_v.at[j]], rows_v, sem).wait()
                for kk in range(C // 16):
                    v = dst_v[j, pl.ds(kk * 16, 16)]
                    if p == 0:
                        idx16 = jnp.minimum(v, NH)
                    else:
                        idx16 = jnp.where(v >= NH, v - NH, NH)
                    midx_v[pl.ds(kk * 16, 16)] = idx16
                pltpu.sync_copy(rows_v, acc_sh.at[midx_v], add=True)
                return carry

            lax.fori_loop(0, CHUNKS, body, 0)
            plsc.subcore_barrier()

            @pl.when(s < NS - 1)
            def _():
                pltpu.sync_copy(acc_sh.at[pl.ds(s * RPA, RPA)],
                                stage_v.at[pl.ds(0, RPA)])
                pltpu.sync_copy(stage_v.at[pl.ds(0, RPA)],
                                out_hbm.at[p, c, pl.ds(s * RPA, RPA)])

            @pl.when(s == NS - 1)
            def _():
                pltpu.sync_copy(acc_sh.at[pl.ds((NS - 1) * RPA, RPA_LAST_W)],
                                stage_v.at[pl.ds(0, RPA_LAST_W)])
                pltpu.sync_copy(stage_v.at[pl.ds(0, RPA_LAST_W)],
                                out_hbm.at[p, c,
                                           pl.ds((NS - 1) * RPA, RPA_LAST_W)])

            plsc.subcore_barrier()

    return k(y, src3, dst3, zeros2d)


def _dinv(degp_ref):
    deg = degp_ref[...][:, 0] + degp_ref[...][:, 1] + 1.0
    return lax.rsqrt(deg)[:, None]


def _tc_pre(x, lin_W, lin_b2, conv1_W, degp_t):
    def body(x_ref, w1_ref, b1_ref, w2_ref, dg_ref, y_ref):
        h = jnp.dot(x_ref[...], w1_ref[...],
                    preferred_element_type=jnp.float32) + b1_ref[...]
        xw = jnp.dot(h, w2_ref[...], preferred_element_type=jnp.float32)
        y_ref[...] = xw * _dinv(dg_ref)

    return pl.pallas_call(
        body,
        grid=(G,),
        in_specs=[
            pl.BlockSpec((BR, D), lambda i: (i, 0)),
            pl.BlockSpec((D, D), lambda i: (0, 0)),
            pl.BlockSpec((1, D), lambda i: (0, 0)),
            pl.BlockSpec((D, D), lambda i: (0, 0)),
            pl.BlockSpec((BR, NC), lambda i: (i, 0)),
        ],
        out_specs=pl.BlockSpec((BR, D), lambda i: (i, 0)),
        out_shape=jax.ShapeDtypeStruct((N, D), jnp.float32),
    )(x, lin_W, lin_b2, conv1_W, degp_t)


def _tc_mid1(p, y1, conv1_b2, degp_t):
    def body(p_ref, y_ref, b_ref, dg_ref, z_ref, st_ref, sacc):
        agg = p_ref[0, 0] + p_ref[0, 1]
        z = (agg + y_ref[...]) * _dinv(dg_ref) + b_ref[...]
        z_ref[...] = z
        i = pl.program_id(0)

        @pl.when(i == 0)
        def _():
            sacc[...] = jnp.zeros_like(sacc)

        sacc[...] += jnp.concatenate(
            [jnp.sum(z, 0, keepdims=True), jnp.sum(z * z, 0, keepdims=True)], 0)

        @pl.when(i == G - 1)
        def _():
            st_ref[...] = sacc[...]

    return pl.pallas_call(
        body,
        grid=(G,),
        in_specs=[
            pl.BlockSpec((1, NC, BR, D), lambda i: (i // 5, 0, i % 5, 0)),
            pl.BlockSpec((BR, D), lambda i: (i, 0)),
            pl.BlockSpec((1, D), lambda i: (0, 0)),
            pl.BlockSpec((BR, NC), lambda i: (i, 0)),
        ],
        out_specs=[
            pl.BlockSpec((BR, D), lambda i: (i, 0)),
            pl.BlockSpec((2, D), lambda i: (0, 0)),
        ],
        out_shape=[
            jax.ShapeDtypeStruct((N, D), jnp.float32),
            jax.ShapeDtypeStruct((2, D), jnp.float32),
        ],
        scratch_shapes=[pltpu.VMEM((2, D), jnp.float32)],
    )(p, y1, conv1_b2, degp_t)


def _tc_mid2(z1, stats, gamma2, beta2, conv2_W, degp_t):
    def body(z_ref, st_ref, g_ref, be_ref, w_ref, dg_ref, y_ref):
        mean = st_ref[...][0:1, :] / float(N)
        var = st_ref[...][1:2, :] / float(N) - mean * mean
        inv = lax.rsqrt(var + 1e-5)
        h = (z_ref[...] - mean) * inv * g_ref[...] + be_ref[...]
        h = jnp.maximum(h, 0.0)
        y = jnp.dot(h, w_ref[...], preferred_element_type=jnp.float32)
        y_ref[...] = y * _dinv(dg_ref)

    return pl.pallas_call(
        body,
        grid=(G,),
        in_specs=[
            pl.BlockSpec((BR, D), lambda i: (i, 0)),
            pl.BlockSpec((2, D), lambda i: (0, 0)),
            pl.BlockSpec((1, D), lambda i: (0, 0)),
            pl.BlockSpec((1, D), lambda i: (0, 0)),
            pl.BlockSpec((D, D), lambda i: (0, 0)),
            pl.BlockSpec((BR, NC), lambda i: (i, 0)),
        ],
        out_specs=pl.BlockSpec((BR, D), lambda i: (i, 0)),
        out_shape=jax.ShapeDtypeStruct((N, D), jnp.float32),
    )(z1, stats, gamma2, beta2, conv2_W, degp_t)


def _tc_fin(p, y2, conv2_b2, degp_t):
    def body(p_ref, y_ref, b_ref, dg_ref, o_ref):
        agg = p_ref[0, 0] + p_ref[0, 1]
        o_ref[...] = (agg + y_ref[...]) * _dinv(dg_ref) + b_ref[...]

    return pl.pallas_call(
        body,
        grid=(G,),
        in_specs=[
            pl.BlockSpec((1, NC, BR, D), lambda i: (i // 5, 0, i % 5, 0)),
            pl.BlockSpec((BR, D), lambda i: (i, 0)),
            pl.BlockSpec((1, D), lambda i: (0, 0)),
            pl.BlockSpec((BR, NC), lambda i: (i, 0)),
        ],
        out_specs=pl.BlockSpec((BR, D), lambda i: (i, 0)),
        out_shape=jax.ShapeDtypeStruct((N, D), jnp.float32),
    )(p, y2, conv2_b2, degp_t)


def kernel(x, edge_index, lin_W, lin_b, conv1_W, conv1_b, bn1_gamma,
           bn1_beta, conv2_W, conv2_b):
    pad = E_PAD - E
    src_p = jnp.concatenate(
        [edge_index[0], jnp.zeros((pad,), edge_index.dtype)])
    dst_p = jnp.concatenate(
        [edge_index[1], jnp.full((pad,), N, edge_index.dtype)])
    src3 = src_p.reshape(NW, CHUNKS, C)
    dst3 = dst_p.reshape(NW, CHUNKS, C)
    zeros2d = jnp.zeros((RPA_LAST_I, D), jnp.float32)
    zeros1d = jnp.zeros((DPS,), jnp.float32)
    ones1d = jnp.ones((C,), jnp.float32)

    degp = _sc_deg(dst3, zeros1d, ones1d)
    degp_t = degp.reshape(NC, DEG_PAD)[:, :N].T

    y1 = _tc_pre(x, lin_W, lin_b.reshape(1, D), conv1_W, degp_t)
    p1 = _sc_agg(y1, src3, dst3, zeros2d)
    z1, stats = _tc_mid1(p1, y1, conv1_b.reshape(1, D), degp_t)
    y2 = _tc_mid2(z1, stats, bn1_gamma.reshape(1, D),
                  bn1_beta.reshape(1, D), conv2_W, degp_t)
    p2 = _sc_agg(y2, src3, dst3, zeros2d)
    out = _tc_fin(p2, y2, conv2_b.reshape(1, D), degp_t)
    return out


# speedup vs baseline: 7.8306x; 1.1973x over previous
import functools

import jax
import jax.numpy as jnp
from jax import lax
from jax.experimental import pallas as pl
from jax.experimental.pallas import tpu as pltpu
from jax.experimental.pallas import tpu_sc as plsc

N = 10000
D = 128
E = 320000
NC = 2
NS = 16
NW = NC * NS
C = 128
CHUNKS = 79
EPW = C * CHUNKS
E_PAD = NW * EPW
NH = 5000
AR = 5008
RPA = 312
RPA_LAST_W = NH - (NS - 1) * RPA
RPA_LAST_I = AR - (NS - 1) * RPA
DEG_PAD = EPW
DPS = 632
BR = 1000
G = N // BR


def _sc_mesh():
    return plsc.VectorSubcoreMesh(core_axis_name="c", subcore_axis_name="s")


def _sc_deg(dst3, zeros1d, ones1d):
    @functools.partial(
        pl.kernel,
        mesh=_sc_mesh(),
        out_type=jax.ShapeDtypeStruct((NC, NS, DPS), jnp.float32),
        scratch_types=[
            pltpu.VMEM((CHUNKS, C), jnp.int32),
            pltpu.VMEM((C,), jnp.float32),
            pltpu.VMEM((DPS,), jnp.float32),
            pltpu.VMEM_SHARED((DEG_PAD,), jnp.float32),
        ],
    )
    def k(dst_hbm, z_hbm, ones_hbm, out_hbm, dst_v, ones_v, buf_v, deg_sh):
        c = lax.axis_index("c")
        s = lax.axis_index("s")
        wid = s * NC + c
        pltpu.sync_copy(dst_hbm.at[wid], dst_v)
        pltpu.sync_copy(ones_hbm, ones_v)
        pltpu.sync_copy(z_hbm, buf_v)
        pltpu.sync_copy(buf_v, deg_sh.at[pl.ds(s * DPS, DPS)])
        plsc.subcore_barrier()

        def body(j, carry):
            pltpu.sync_copy(ones_v, deg_sh.at[dst_v.at[j]], add=True)
            return carry

        lax.fori_loop(0, CHUNKS, body, 0)
        plsc.subcore_barrier()
        pltpu.sync_copy(deg_sh.at[pl.ds(s * DPS, DPS)], buf_v)
        pltpu.sync_copy(buf_v, out_hbm.at[c, s])

    return k(dst3, zeros1d, ones1d)


def _sc_agg(y, src3, dst3, zeros2d):
    @functools.partial(
        pl.kernel,
        mesh=_sc_mesh(),
        out_type=jax.ShapeDtypeStruct((2, NC, NH, D), jnp.float32),
        scratch_types=[
            pltpu.VMEM((CHUNKS, C), jnp.int32),
            pltpu.VMEM((CHUNKS, C), jnp.int32),
            pltpu.VMEM((C,), jnp.int32),
            pltpu.VMEM((2, C, D), jnp.float32),
            pltpu.VMEM_SHARED((AR, D), jnp.float32),
            pltpu.SemaphoreType.DMA,
            pltpu.SemaphoreType.DMA,
        ],
    )
    def k(y_hbm, src_hbm, dst_hbm, z_hbm, out_hbm, src_v, dst_v, midx_v,
          rows_v, acc_sh, sem0, sem1):
        sems = (sem0, sem1)
        c = lax.axis_index("c")
        s = lax.axis_index("s")
        pltpu.sync_copy(src_hbm.at[c * NS + s], src_v)
        pltpu.sync_copy(dst_hbm.at[c * NS + s], dst_v)

        for p in (0, 1):
            pltpu.sync_copy(z_hbm, rows_v.at[0])

            @pl.when(s < NS - 1)
            def _():
                for off, sz in ((0, C), (C, C), (2 * C, RPA - 2 * C)):
                    pltpu.sync_copy(rows_v.at[0, pl.ds(0, sz)],
                                    acc_sh.at[pl.ds(s * RPA + off, sz)])

            @pl.when(s == NS - 1)
            def _():
                for off, sz in ((0, C), (C, C), (2 * C, RPA_LAST_I - 2 * C)):
                    pltpu.sync_copy(rows_v.at[0, pl.ds(0, sz)],
                                    acc_sh.at[pl.ds(s * RPA + off, sz)])

            plsc.subcore_barrier()

            def _issue(j, slot):
                pltpu.async_copy(y_hbm.at[src_v.at[j]], rows_v.at[slot],
                                 sems[slot])

            def _process(j, slot):
                for kk in range(C // 16):
                    v = dst_v[j, pl.ds(kk * 16, 16)]
                    if p == 0:
                        idx16 = jnp.minimum(v, NH)
                    else:
                        idx16 = jnp.where(v >= NH, v - NH, NH)
                    midx_v[pl.ds(kk * 16, 16)] = idx16
                pltpu.make_async_copy(y_hbm.at[src_v.at[j]], rows_v.at[slot],
                                      sems[slot]).wait()
                pltpu.sync_copy(rows_v.at[slot], acc_sh.at[midx_v], add=True)

                @pl.when(j + 2 < CHUNKS)
                def _():
                    _issue(j + 2, slot)

            _issue(0, 0)
            _issue(1, 1)

            def body(j2, carry):
                a = j2 * 2
                _process(a, 0)

                @pl.when(a + 1 < CHUNKS)
                def _():
                    _process(a + 1, 1)

                return carry

            lax.fori_loop(0, (CHUNKS + 1) // 2, body, 0)
            plsc.subcore_barrier()

            @pl.when(s < NS - 1)
            def _():
                for off, sz in ((0, C), (C, C), (2 * C, RPA - 2 * C)):
                    pltpu.sync_copy(acc_sh.at[pl.ds(s * RPA + off, sz)],
                                    rows_v.at[1, pl.ds(0, sz)])
                    pltpu.sync_copy(rows_v.at[1, pl.ds(0, sz)],
                                    out_hbm.at[p, c, pl.ds(s * RPA + off, sz)])

            @pl.when(s == NS - 1)
            def _():
                for off, sz in ((0, C), (C, C), (2 * C, RPA_LAST_W - 2 * C)):
                    pltpu.sync_copy(acc_sh.at[pl.ds(s * RPA + off, sz)],
                                    rows_v.at[1, pl.ds(0, sz)])
                    pltpu.sync_copy(rows_v.at[1, pl.ds(0, sz)],
                                    out_hbm.at[p, c, pl.ds(s * RPA + off, sz)])

            plsc.subcore_barrier()

    return k(y, src3, dst3, zeros2d)


def _dinv(degp_ref):
    deg = degp_ref[...][:, 0] + degp_ref[...][:, 1] + 1.0
    return lax.rsqrt(deg)[:, None]


def _tc_pre(x, lin_W, lin_b2, conv1_W, degp_t):
    def body(x_ref, w1_ref, b1_ref, w2_ref, dg_ref, y_ref):
        h = jnp.dot(x_ref[...], w1_ref[...],
                    preferred_element_type=jnp.float32) + b1_ref[...]
        xw = jnp.dot(h, w2_ref[...], preferred_element_type=jnp.float32)
        y_ref[...] = xw * _dinv(dg_ref)

    return pl.pallas_call(
        body,
        grid=(G,),
        in_specs=[
            pl.BlockSpec((BR, D), lambda i: (i, 0)),
            pl.BlockSpec((D, D), lambda i: (0, 0)),
            pl.BlockSpec((1, D), lambda i: (0, 0)),
            pl.BlockSpec((D, D), lambda i: (0, 0)),
            pl.BlockSpec((BR, NC), lambda i: (i, 0)),
        ],
        out_specs=pl.BlockSpec((BR, D), lambda i: (i, 0)),
        out_shape=jax.ShapeDtypeStruct((N, D), jnp.float32),
    )(x, lin_W, lin_b2, conv1_W, degp_t)


def _tc_mid1(p, y1, conv1_b2, degp_t):
    def body(p_ref, y_ref, b_ref, dg_ref, z_ref, st_ref, sacc):
        agg = p_ref[0, 0] + p_ref[0, 1]
        z = (agg + y_ref[...]) * _dinv(dg_ref) + b_ref[...]
        z_ref[...] = z
        i = pl.program_id(0)

        @pl.when(i == 0)
        def _():
            sacc[...] = jnp.zeros_like(sacc)

        sacc[...] += jnp.concatenate(
            [jnp.sum(z, 0, keepdims=True), jnp.sum(z * z, 0, keepdims=True)], 0)

        @pl.when(i == G - 1)
        def _():
            st_ref[...] = sacc[...]

    return pl.pallas_call(
        body,
        grid=(G,),
        in_specs=[
            pl.BlockSpec((1, NC, BR, D), lambda i: (i // 5, 0, i % 5, 0)),
            pl.BlockSpec((BR, D), lambda i: (i, 0)),
            pl.BlockSpec((1, D), lambda i: (0, 0)),
            pl.BlockSpec((BR, NC), lambda i: (i, 0)),
        ],
        out_specs=[
            pl.BlockSpec((BR, D), lambda i: (i, 0)),
            pl.BlockSpec((2, D), lambda i: (0, 0)),
        ],
        out_shape=[
            jax.ShapeDtypeStruct((N, D), jnp.float32),
            jax.ShapeDtypeStruct((2, D), jnp.float32),
        ],
        scratch_shapes=[pltpu.VMEM((2, D), jnp.float32)],
    )(p, y1, conv1_b2, degp_t)


def _tc_mid2(z1, stats, gamma2, beta2, conv2_W, degp_t):
    def body(z_ref, st_ref, g_ref, be_ref, w_ref, dg_ref, y_ref):
        mean = st_ref[...][0:1, :] / float(N)
        var = st_ref[...][1:2, :] / float(N) - mean * mean
        inv = lax.rsqrt(var + 1e-5)
        h = (z_ref[...] - mean) * inv * g_ref[...] + be_ref[...]
        h = jnp.maximum(h, 0.0)
        y = jnp.dot(h, w_ref[...], preferred_element_type=jnp.float32)
        y_ref[...] = y * _dinv(dg_ref)

    return pl.pallas_call(
        body,
        grid=(G,),
        in_specs=[
            pl.BlockSpec((BR, D), lambda i: (i, 0)),
            pl.BlockSpec((2, D), lambda i: (0, 0)),
            pl.BlockSpec((1, D), lambda i: (0, 0)),
            pl.BlockSpec((1, D), lambda i: (0, 0)),
            pl.BlockSpec((D, D), lambda i: (0, 0)),
            pl.BlockSpec((BR, NC), lambda i: (i, 0)),
        ],
        out_specs=pl.BlockSpec((BR, D), lambda i: (i, 0)),
        out_shape=jax.ShapeDtypeStruct((N, D), jnp.float32),
    )(z1, stats, gamma2, beta2, conv2_W, degp_t)


def _tc_fin(p, y2, conv2_b2, degp_t):
    def body(p_ref, y_ref, b_ref, dg_ref, o_ref):
        agg = p_ref[0, 0] + p_ref[0, 1]
        o_ref[...] = (agg + y_ref[...]) * _dinv(dg_ref) + b_ref[...]

    return pl.pallas_call(
        body,
        grid=(G,),
        in_specs=[
            pl.BlockSpec((1, NC, BR, D), lambda i: (i // 5, 0, i % 5, 0)),
            pl.BlockSpec((BR, D), lambda i: (i, 0)),
            pl.BlockSpec((1, D), lambda i: (0, 0)),
            pl.BlockSpec((BR, NC), lambda i: (i, 0)),
        ],
        out_specs=pl.BlockSpec((BR, D), lambda i: (i, 0)),
        out_shape=jax.ShapeDtypeStruct((N, D), jnp.float32),
    )(p, y2, conv2_b2, degp_t)


def kernel(x, edge_index, lin_W, lin_b, conv1_W, conv1_b, bn1_gamma,
           bn1_beta, conv2_W, conv2_b):
    pad = E_PAD - E
    src_p = jnp.concatenate(
        [edge_index[0], jnp.zeros((pad,), edge_index.dtype)])
    dst_p = jnp.concatenate(
        [edge_index[1], jnp.full((pad,), N, edge_index.dtype)])
    src3 = src_p.reshape(NW, CHUNKS, C)
    dst3 = dst_p.reshape(NW, CHUNKS, C)
    zeros2d = jnp.zeros((C, D), jnp.float32)
    zeros1d = jnp.zeros((DPS,), jnp.float32)
    ones1d = jnp.ones((C,), jnp.float32)

    degp = _sc_deg(dst3, zeros1d, ones1d)
    degp_t = degp.reshape(NC, DEG_PAD)[:, :N].T

    y1 = _tc_pre(x, lin_W, lin_b.reshape(1, D), conv1_W, degp_t)
    p1 = _sc_agg(y1, src3, dst3, zeros2d)
    z1, stats = _tc_mid1(p1, y1, conv1_b.reshape(1, D), degp_t)
    y2 = _tc_mid2(z1, stats, bn1_gamma.reshape(1, D),
                  bn1_beta.reshape(1, D), conv2_W, degp_t)
    p2 = _sc_agg(y2, src3, dst3, zeros2d)
    out = _tc_fin(p2, y2, conv2_b.reshape(1, D), degp_t)
    return out
